# baseline (speedup 1.0000x reference)
import jax
import jax.numpy as jnp
from jax.experimental import pallas as pl
from jax.experimental.pallas import tpu as pltpu

N = 2048
D = 64
R1 = 512
R2 = 256
TK = 10
TH = 5
_DNT = (((1,), (1,)), ((), ()))
_DN = (((1,), (0,)), ((), ()))
F32 = jnp.float32
BF16 = jnp.bfloat16


def _split(x):
    hi = x.astype(BF16)
    lo = (x - hi.astype(F32)).astype(BF16)
    return hi, lo


def _dot3_t(a, b):
    ahi, alo = _split(a)
    bhi, blo = _split(b)
    out = jax.lax.dot_general(ahi, bhi, _DNT, preferred_element_type=F32)
    out = out + jax.lax.dot_general(ahi, blo, _DNT, preferred_element_type=F32)
    out = out + jax.lax.dot_general(alo, bhi, _DNT, preferred_element_type=F32)
    return out


def _sq_row(x):
    xt = jnp.transpose(x)
    return jnp.sum(xt * xt, axis=0, keepdims=True)


def _sq_col(x):
    return jnp.sum(x * x, axis=1, keepdims=True)


def _dist_tile(xb, xf):
    d2 = _sq_col(xb) + _sq_row(xf) - 2.0 * _dot3_t(xb, xf)
    return jnp.sqrt(jnp.maximum(d2, 1e-12))


def _norm_rows(x):
    n = jnp.sqrt(jnp.sum(x * x, axis=1, keepdims=True))
    return x / jnp.maximum(n, 1e-12)


def _mega(t_ref, ic_ref, ir_ref, f_ref, g_ref,
          rcf_ref, rcg_ref, kl_ref,
          a_ref, b_ref, h_ref, d_ref, e_ref):
    i = pl.program_id(0)

    @pl.when(i < 4)
    def _s2():
        blk = i * R1
        tnb = _norm_rows(t_ref[pl.ds(blk, R1), :])
        tnf = _norm_rows(t_ref[...])
        d2 = jnp.maximum(
            _sq_col(tnb) + _sq_row(tnf) - 2.0 * _dot3_t(tnb, tnf), 1e-12)
        wp = jnp.exp(-d2)
        same = ic_ref[pl.ds(blk, R1), :] == ir_ref[...]
        wpc = jnp.where(same, 1.0, wp)
        iota = jax.lax.broadcasted_iota(jnp.int32, (R1, N), 1)
        h5 = jnp.zeros((R1, N), jnp.bool_)
        for it in range(TK):
            j = jnp.argmax(wpc, axis=1, keepdims=True).astype(jnp.int32)
            wpc = jnp.where(iota == j, -1.0, wpc)
            if it == TH - 1:
                h5 = wpc < 0.0
        accf = (wpc < 0.0).astype(F32)
        a_ref[pl.ds(blk, R1), :] = accf.astype(BF16)
        b_ref[:, pl.ds(blk, R1)] = jnp.transpose(accf).astype(BF16)
        h_ref[pl.ds(blk, R1), :] = h5.astype(BF16)

    @pl.when(i == 4)
    def _s3a():
        a_ref[...] = a_ref[...] * b_ref[...]

    @pl.when(jnp.logical_and(i >= 5, i < 9))
    def _s3b():
        blk = (i - 5) * R1
        vb = a_ref[pl.ds(blk, R1), :]
        vv = jax.lax.dot_general(vb, a_ref[...], _DNT,
                                 preferred_element_type=F32)
        rc = jnp.sum(vb.astype(F32), axis=1, keepdims=True)
        wct = vb.astype(F32) * vv / jnp.maximum(rc, 1.0)
        b_ref[pl.ds(blk, R1), :] = wct.astype(BF16)

    @pl.when(jnp.logical_and(i >= 9, i < 13))
    def _sa():
        blk = (i - 9) * R1
        w = jax.lax.dot_general(h_ref[pl.ds(blk, R1), :], b_ref[...], _DN,
                                preferred_element_type=F32) * (1.0 / TH)
        d_ref[pl.ds(blk, R1), :] = w.astype(BF16)
        e_ref[:, pl.ds(blk, R1)] = jnp.transpose(w).astype(BF16)

    @pl.when(i >= 13)
    def _sb():
        k = i - 13
        blk = k * R2
        tnb = _norm_rows(t_ref[pl.ds(blk, R2), :])
        tnf = _norm_rows(t_ref[...])
        d2 = jnp.maximum(
            _sq_col(tnb) + _sq_row(tnf) - 2.0 * _dot3_t(tnb, tnf), 1e-12)
        wp = jnp.exp(-d2)
        wc = 0.5 * (d_ref[pl.ds(blk, R2), :].astype(F32)
                    + e_ref[pl.ds(blk, R2), :].astype(F32))
        w = 0.5 * (wp + wc)
        rows = blk + jax.lax.broadcasted_iota(jnp.int32, (R2, N), 0)
        cols = jax.lax.broadcasted_iota(jnp.int32, (R2, N), 1)
        offd = (rows != cols).astype(F32)
        wo = w * offd
        wo2 = offd - wo

        def terms(sb_, sfull):
            s = _dist_tile(sb_, sfull)
            rs = jnp.sum(s, axis=1, keepdims=True)
            sn = s * (N / rs)
            es = jnp.exp(-sn)
            se = jnp.sum(es, axis=1, keepdims=True)
            lse = jnp.log(se)
            hh = jnp.maximum(1.0 - sn, 0.0)
            loss = jnp.sum(sn * sn * wo + hh * hh * wo2, keepdims=True)
            return loss, sn, lse, es, se

        lf, sfn, lsea, _, _ = terms(f_ref[pl.ds(blk, R2), :], f_ref[...])
        lg, sgn, lseb, eg, seg = terms(g_ref[pl.ds(blk, R2), :], g_ref[...])
        p = eg * (1.0 / seg)
        kl = (jnp.sum(p * (sfn - sgn), keepdims=True)
              + jnp.sum(lsea - lseb, keepdims=True))

        @pl.when(k == 0)
        def _init():
            rcf_ref[...] = jnp.zeros((1, 1), F32)
            rcg_ref[...] = jnp.zeros((1, 1), F32)
            kl_ref[...] = jnp.zeros((1, 1), F32)

        rcf_ref[...] += lf
        rcg_ref[...] += lg
        kl_ref[...] += kl


def _00(i):
    return (0, 0)


def kernel(s_f, s_g, t_g, idx):
    idx = idx.astype(jnp.int32)
    idxc = idx.reshape(N, 1)
    idxr = idx.reshape(1, N)

    ef = pl.BlockSpec((N, D), _00)
    one = pl.BlockSpec((1, 1), _00)
    rcf, rcg, kl = pl.pallas_call(
        _mega,
        grid=(21,),
        in_specs=[ef, pl.BlockSpec((N, 1), _00), pl.BlockSpec((1, N), _00),
                  ef, ef],
        out_specs=[one, one, one],
        out_shape=[jax.ShapeDtypeStruct((1, 1), F32)] * 3,
        scratch_shapes=[pltpu.VMEM((N, N), BF16)] * 5,
        compiler_params=pltpu.CompilerParams(vmem_limit_bytes=58 * 2**20),
    )(t_g, idxc, idxr, s_f, s_g)

    scale = 1.0 / (N * (N - 1))
    loss_rc = 0.5 * (rcf[0, 0] + rcg[0, 0]) * scale
    loss_kl = kl[0, 0] / N
    return (loss_rc, loss_kl, loss_rc + loss_kl)
